# speedup vs baseline: 5.7207x; 5.7207x over previous
import math

import jax
import jax.numpy as jnp
from jax import lax
from jax.experimental import pallas as pl
from jax.experimental.pallas import tpu as pltpu
from jax.experimental.pallas import tpu_sc as plsc

EMB_D = 128
SCALE = math.sqrt(float(EMB_D))

NC = 2
NS = 16
NW = NC * NS

CH = 128


def _scale_body(w_ref, o_ref):
    o_ref[...] = w_ref[...] * SCALE


def _scale_table(w):
    v, d = w.shape
    br = 2000
    assert v % br == 0
    return pl.pallas_call(
        _scale_body,
        grid=(v // br,),
        in_specs=[pl.BlockSpec((br, d), lambda i: (i, 0))],
        out_specs=pl.BlockSpec((br, d), lambda i: (i, 0)),
        out_shape=jax.ShapeDtypeStruct((v, d), w.dtype),
    )(w)


def _make_gather(nch):
    mesh = plsc.VectorSubcoreMesh(
        core_axis_name="c", subcore_axis_name="s",
        num_cores=NC, num_subcores=NS,
    )

    def body(table_hbm, tok_hbm, out_hbm, idx_v, buf, sem):
        wid = lax.axis_index("s") * NC + lax.axis_index("c")
        pltpu.sync_copy(tok_hbm.at[wid], idx_v)

        @pl.loop(0, nch)
        def _chunk(g):
            pltpu.async_copy(table_hbm.at[idx_v.at[g]], buf, sem).wait()
            pltpu.sync_copy(buf, out_hbm.at[wid, g])

    return pl.kernel(
        body,
        out_type=jax.ShapeDtypeStruct((NW, nch, CH, EMB_D), jnp.float32),
        mesh=mesh,
        scratch_types=[
            pltpu.VMEM((nch, CH), jnp.int32),
            pltpu.VMEM((CH, EMB_D), jnp.float32),
            pltpu.SemaphoreType.DMA,
        ],
    )


def kernel(tokens, embedding_weight):
    batch, seq = tokens.shape
    total = batch * seq
    assert total % (NW * CH) == 0
    nch = total // (NW * CH)

    scaled = _scale_table(embedding_weight)
    tok = tokens.reshape(NW, nch, CH).astype(jnp.int32)
    out = _make_gather(nch)(scaled, tok)
    return out.reshape(batch, seq, EMB_D)


# speedup vs baseline: 7.9921x; 1.3970x over previous
import math

import jax
import jax.numpy as jnp
from jax import lax
from jax.experimental import pallas as pl
from jax.experimental.pallas import tpu as pltpu
from jax.experimental.pallas import tpu_sc as plsc

EMB_D = 128
SCALE = math.sqrt(float(EMB_D))

NC = 2
NS = 16
NW = NC * NS

CH = 128


def _scale_body(w_ref, o_ref):
    o_ref[...] = w_ref[...] * SCALE


def _scale_table(w):
    v, d = w.shape
    br = 2000
    assert v % br == 0
    return pl.pallas_call(
        _scale_body,
        grid=(v // br,),
        in_specs=[pl.BlockSpec((br, d), lambda i: (i, 0))],
        out_specs=pl.BlockSpec((br, d), lambda i: (i, 0)),
        out_shape=jax.ShapeDtypeStruct((v, d), w.dtype),
    )(w)


NBUF = 4
LOOK = 2


def _make_gather(nch):
    mesh = plsc.VectorSubcoreMesh(
        core_axis_name="c", subcore_axis_name="s",
        num_cores=NC, num_subcores=NS,
    )

    def body(table_hbm, tok_hbm, out_hbm, idx_v, *rest):
        bufs = rest[:NBUF]
        gsems = rest[NBUF:2 * NBUF]
        wsems = rest[2 * NBUF:3 * NBUF]
        wid = lax.axis_index("s") * NC + lax.axis_index("c")
        pltpu.sync_copy(tok_hbm.at[wid], idx_v)

        for c in range(LOOK):
            pltpu.async_copy(table_hbm.at[idx_v.at[c]], bufs[c], gsems[c])

        @pl.loop(0, nch, step=NBUF)
        def _pass(g):
            for b in range(NBUF):
                c = g + b
                s2 = (b + LOOK) % NBUF

                @pl.when(c + LOOK < nch)
                def _():
                    @pl.when(c + LOOK >= NBUF)
                    def _():
                        pltpu.make_async_copy(
                            bufs[s2], out_hbm.at[wid, 0], wsems[s2]).wait()
                    pltpu.async_copy(
                        table_hbm.at[idx_v.at[c + LOOK]], bufs[s2], gsems[s2])

                pltpu.make_async_copy(
                    table_hbm.at[idx_v.at[c]], bufs[b], gsems[b]).wait()
                pltpu.async_copy(bufs[b], out_hbm.at[wid, c], wsems[b])

        for b in range(NBUF):
            pltpu.make_async_copy(bufs[b], out_hbm.at[wid, 0], wsems[b]).wait()

    return pl.kernel(
        body,
        out_type=jax.ShapeDtypeStruct((NW, nch, CH, EMB_D), jnp.float32),
        mesh=mesh,
        scratch_types=[
            pltpu.VMEM((nch, CH), jnp.int32),
            *[pltpu.VMEM((CH, EMB_D), jnp.float32) for _ in range(NBUF)],
            *[pltpu.SemaphoreType.DMA for _ in range(2 * NBUF)],
        ],
    )


def kernel(tokens, embedding_weight):
    batch, seq = tokens.shape
    total = batch * seq
    assert total % (NW * CH) == 0
    nch = total // (NW * CH)

    scaled = _scale_table(embedding_weight)
    tok = tokens.reshape(NW, nch, CH).astype(jnp.int32)
    out = _make_gather(nch)(scaled, tok)
    return out.reshape(batch, seq, EMB_D)
